# baseline (speedup 1.0000x reference)
import functools
import math

import jax
import jax.numpy as jnp
from jax import lax
from jax.experimental import pallas as pl
from jax.experimental.pallas import tpu as pltpu
from jax.experimental.pallas import tpu_sc as plsc

N = 10000
NP = 5000
D = 256
DH = 256
DO = 128
E = 160000
EL = E + N
NPAD = 10240
RPT = NPAD // 16
CW = 125
GN = 44
NCH = 88
EP = 16 * NCH * CW
NCH2 = 44
DCH = 42
DCW = 128
EPD = 32 * DCH * DCW
BN_K = 1.0 / math.sqrt(1.0 + 1e-5)

_MESH = plsc.VectorSubcoreMesh(core_axis_name="c", subcore_axis_name="s")


def _edge_pipe(nch, h_hbm, sv, dv, ga, gb, acc, sg0, sg1, ss0, ss1):
  pltpu.async_copy(h_hbm.at[sv.at[0]], ga, sg0)
  pltpu.async_copy(h_hbm.at[sv.at[1]], gb, sg1)

  @pl.loop(0, nch // 2 - 1)
  def _(i):
    j = 2 * i
    pltpu.make_async_copy(h_hbm.at[sv.at[j]], ga, sg0).wait()
    pltpu.async_copy(ga, acc.at[dv.at[j]], ss0, add=True)
    pltpu.make_async_copy(h_hbm.at[sv.at[j + 1]], gb, sg1).wait()
    pltpu.async_copy(gb, acc.at[dv.at[j + 1]], ss1, add=True)
    pltpu.make_async_copy(ga, acc.at[dv.at[j]], ss0).wait()
    pltpu.async_copy(h_hbm.at[sv.at[j + 2]], ga, sg0)
    pltpu.make_async_copy(gb, acc.at[dv.at[j + 1]], ss1).wait()
    pltpu.async_copy(h_hbm.at[sv.at[j + 3]], gb, sg1)

  pltpu.make_async_copy(h_hbm.at[sv.at[nch - 2]], ga, sg0).wait()
  pltpu.async_copy(ga, acc.at[dv.at[nch - 2]], ss0, add=True)
  pltpu.make_async_copy(h_hbm.at[sv.at[nch - 1]], gb, sg1).wait()
  pltpu.async_copy(gb, acc.at[dv.at[nch - 1]], ss1, add=True)
  pltpu.make_async_copy(ga, acc.at[dv.at[nch - 2]], ss0).wait()
  pltpu.make_async_copy(gb, acc.at[dv.at[nch - 1]], ss1).wait()


def _zero_acc(gbuf, acc, s):
  zv = jnp.zeros((16,), jnp.float32)

  @pl.loop(0, 16)
  def _(r):
    for j in range(8):
      gbuf[r, pl.ds(j * 16, 16)] = zv

  @pl.loop(0, RPT // 16)
  def _(t):
    pltpu.sync_copy(gbuf.at[pl.ds(0, 16)], acc.at[pl.ds(s * RPT + t * 16, 16)])


@functools.partial(
    pl.kernel,
    out_type=jax.ShapeDtypeStruct((2, NPAD, 128), jnp.float32),
    mesh=_MESH,
    scratch_types=[
        pltpu.VMEM((GN, CW), jnp.int32),
        pltpu.VMEM((GN, CW), jnp.int32),
        pltpu.VMEM((CW, 128), jnp.float32),
        pltpu.VMEM((CW, 128), jnp.float32),
        pltpu.VMEM_SHARED((NPAD, 128), jnp.float32),
        pltpu.SemaphoreType.DMA,
        pltpu.SemaphoreType.DMA,
        pltpu.SemaphoreType.DMA,
        pltpu.SemaphoreType.DMA,
    ],
    name="agg1",
)
def _agg1(h_hbm, src_hbm, dst_hbm, out_hbm, src_v, dst_v, ga, gb, acc,
          sg0, sg1, ss0, ss1):
  c = lax.axis_index("c")
  s = lax.axis_index("s")
  _zero_acc(ga, acc, s)
  plsc.subcore_barrier()
  for g in range(NCH // GN):
    pltpu.sync_copy(src_hbm.at[c, s, g], src_v)
    pltpu.sync_copy(dst_hbm.at[s, g], dst_v)
    _edge_pipe(GN, h_hbm, src_v, dst_v, ga, gb, acc, sg0, sg1, ss0, ss1)
  plsc.subcore_barrier()
  pltpu.sync_copy(acc.at[pl.ds(s * RPT, RPT)],
                  out_hbm.at[c, pl.ds(s * RPT, RPT)])


@functools.partial(
    pl.kernel,
    out_type=jax.ShapeDtypeStruct((2, NPAD, 128), jnp.float32),
    mesh=_MESH,
    scratch_types=[
        pltpu.VMEM((NCH2, CW), jnp.int32),
        pltpu.VMEM((NCH2, CW), jnp.int32),
        pltpu.VMEM((CW, 128), jnp.float32),
        pltpu.VMEM((CW, 128), jnp.float32),
        pltpu.VMEM_SHARED((NPAD, 128), jnp.float32),
        pltpu.SemaphoreType.DMA,
        pltpu.SemaphoreType.DMA,
        pltpu.SemaphoreType.DMA,
        pltpu.SemaphoreType.DMA,
    ],
    name="agg2",
)
def _agg2(h_hbm, src_hbm, dst_hbm, out_hbm, src_v, dst_v, ga, gb, acc,
          sg0, sg1, ss0, ss1):
  c = lax.axis_index("c")
  s = lax.axis_index("s")
  w = c * 16 + s
  _zero_acc(ga, acc, s)
  pltpu.sync_copy(src_hbm.at[w], src_v)
  pltpu.sync_copy(dst_hbm.at[w], dst_v)
  plsc.subcore_barrier()
  _edge_pipe(NCH2, h_hbm, src_v, dst_v, ga, gb, acc, sg0, sg1, ss0, ss1)
  plsc.subcore_barrier()
  pltpu.sync_copy(acc.at[pl.ds(s * RPT, RPT)],
                  out_hbm.at[c, pl.ds(s * RPT, RPT)])


@functools.partial(
    pl.kernel,
    out_type=jax.ShapeDtypeStruct((2, NPAD, 128), jnp.float32),
    mesh=_MESH,
    scratch_types=[
        pltpu.VMEM((DCH, DCW), jnp.int32),
        pltpu.VMEM((DCW, 128), jnp.float32),
        pltpu.VMEM_SHARED((NPAD, 128), jnp.float32),
    ],
    name="deg",
)
def _deg(dst_hbm, out_hbm, dst_v, obuf, acc):
  c = lax.axis_index("c")
  s = lax.axis_index("s")
  w = c * 16 + s
  _zero_acc(obuf, acc, s)
  ones = jnp.ones((16,), jnp.float32)

  @pl.loop(0, DCW)
  def _(r):
    for j in range(8):
      obuf[r, pl.ds(j * 16, 16)] = ones

  pltpu.sync_copy(dst_hbm.at[w], dst_v)
  plsc.subcore_barrier()

  @pl.loop(0, DCH)
  def _(j):
    pltpu.sync_copy(obuf, acc.at[dst_v.at[j]], add=True)

  plsc.subcore_barrier()
  pltpu.sync_copy(acc.at[pl.ds(s * RPT, RPT)],
                  out_hbm.at[c, pl.ds(s * RPT, RPT)])


def _dinv_body(degp_ref, out_ref):
  total = degp_ref[0] + degp_ref[1]
  out_ref[...] = lax.rsqrt(total[:, 0:1])


def _dinv(degp):
  return pl.pallas_call(
      _dinv_body,
      out_shape=jax.ShapeDtypeStruct((NPAD, 1), jnp.float32),
  )(degp)


def _mm1_body(x_ref, w_ref, dinv_ref, out_ref):
  h = jnp.dot(x_ref[...], w_ref[...], preferred_element_type=jnp.float32)
  out_ref[0] = h * dinv_ref[...]


def _mm1(x_pooled, W1, dinv_p):
  blk = 1000
  return pl.pallas_call(
      _mm1_body,
      grid=(2, NP // blk),
      in_specs=[
          pl.BlockSpec((blk, D), lambda c, r: (r, 0)),
          pl.BlockSpec((D, 128), lambda c, r: (0, c)),
          pl.BlockSpec((blk, 1), lambda c, r: (r, 0)),
      ],
      out_specs=pl.BlockSpec((1, blk, 128), lambda c, r: (c, r, 0)),
      out_shape=jax.ShapeDtypeStruct((2, NP, 128), jnp.float32),
  )(x_pooled, W1, dinv_p)


def _tcb_body(a0_ref, a1_ref, dinv_ref, b1_ref, g1_ref, bt_ref, wf_ref,
              h2_ref):
  agg = jnp.concatenate([a0_ref[0], a1_ref[0]], axis=-1)
  dinv = dinv_ref[...]
  z = agg * dinv + b1_ref[...]
  a = jnp.maximum(z * (g1_ref[...] * BN_K) + bt_ref[...], 0.0)
  h2_ref[...] = jnp.dot(a, wf_ref[...],
                        preferred_element_type=jnp.float32) * dinv


def _tcb(agg1, dinv, b1, gamma1, beta1, Wf):
  blk = 1000
  return pl.pallas_call(
      _tcb_body,
      grid=(N // blk,),
      in_specs=[
          pl.BlockSpec((1, blk, 128), lambda r: (0, r, 0)),
          pl.BlockSpec((1, blk, 128), lambda r: (1, r, 0)),
          pl.BlockSpec((blk, 1), lambda r: (r, 0)),
          pl.BlockSpec((1, DH), lambda r: (0, 0)),
          pl.BlockSpec((1, DH), lambda r: (0, 0)),
          pl.BlockSpec((1, DH), lambda r: (0, 0)),
          pl.BlockSpec((DH, DO), lambda r: (0, 0)),
      ],
      out_specs=pl.BlockSpec((blk, DO), lambda r: (r, 0)),
      out_shape=jax.ShapeDtypeStruct((N, DO), jnp.float32),
  )(agg1, agg1, dinv, b1, gamma1, beta1, Wf)


def _tcc_body(p0_ref, p1_ref, dinv_ref, bf_ref, out_ref):
  agg = p0_ref[0] + p1_ref[0]
  out_ref[...] = agg * dinv_ref[...] + bf_ref[...]


def _tcc(agg2, dinv, bf):
  blk = 1000
  return pl.pallas_call(
      _tcc_body,
      grid=(N // blk,),
      in_specs=[
          pl.BlockSpec((1, blk, 128), lambda r: (0, r, 0)),
          pl.BlockSpec((1, blk, 128), lambda r: (1, r, 0)),
          pl.BlockSpec((blk, 1), lambda r: (r, 0)),
          pl.BlockSpec((1, DO), lambda r: (0, 0)),
      ],
      out_specs=pl.BlockSpec((blk, DO), lambda r: (r, 0)),
      out_shape=jax.ShapeDtypeStruct((N, DO), jnp.float32),
  )(agg2, agg2, dinv, bf)


def kernel(x_pooled, edge_index_latent, batch_latent, perm, edge_index_full,
           batch_full, num_nodes_before_pool, W1, b1, gamma1, beta1, Wf, bf):
  src = edge_index_full[0].astype(jnp.int32)
  dst = edge_index_full[1].astype(jnp.int32)
  loops = jnp.arange(N, dtype=jnp.int32)
  src_l = jnp.concatenate([src, loops])
  dst_l = jnp.concatenate([dst, loops])
  src_p1 = jnp.concatenate(
      [src_l, jnp.full((EP - EL,), NP, jnp.int32)]).reshape(
          16, NCH // GN, GN, CW)
  src2 = jnp.stack([src_p1, src_p1 + N])
  src_c2 = jnp.concatenate(
      [src_l, jnp.full((EP - EL,), N, jnp.int32)]).reshape(32, NCH2, CW)
  dst_p = jnp.concatenate(
      [dst_l, jnp.full((EP - EL,), N, jnp.int32)])
  dst16 = dst_p.reshape(16, NCH // GN, GN, CW)
  dst32 = dst_p.reshape(32, NCH2, CW)
  dst_deg = jnp.concatenate(
      [dst_l, jnp.full((EPD - EL,), N, jnp.int32)]).reshape(32, DCH, DCW)

  degp = _deg(dst_deg)
  dinv = _dinv(degp)

  hs = _mm1(x_pooled, W1, dinv[:NP])
  ht1 = jnp.concatenate(
      [hs, jnp.zeros((2, NP, 128), jnp.float32)], axis=1).reshape(2 * N, 128)

  agg1 = _agg1(ht1, src2, dst16)

  h2 = _tcb(agg1, dinv[:N], b1.reshape(1, DH), gamma1.reshape(1, DH),
            beta1.reshape(1, DH), Wf)
  ht2 = jnp.concatenate([h2, jnp.zeros((8, DO), jnp.float32)], axis=0)

  agg2 = _agg2(ht2, src_c2, dst32)

  out = _tcc(agg2, dinv[:N], bf.reshape(1, DO))
  return out, batch_full


# speedup vs baseline: 1.0330x; 1.0330x over previous
import functools
import math

import jax
import jax.numpy as jnp
from jax import lax
from jax.experimental import pallas as pl
from jax.experimental.pallas import tpu as pltpu
from jax.experimental.pallas import tpu_sc as plsc

N = 10000
NP = 5000
D = 256
DH = 256
DO = 128
E = 160000
EL = E + N
NPAD = 10240
RPT = NPAD // 16
CW = 125
GN = 44
NCH = 88
EP = 16 * NCH * CW
NCH2 = 44
DCH = 42
DCW = 128
EPD = 32 * DCH * DCW
BN_K = 1.0 / math.sqrt(1.0 + 1e-5)

_MESH = plsc.VectorSubcoreMesh(core_axis_name="c", subcore_axis_name="s")


def _edge_pipe(nch, h_hbm, sv, dv, ga, gb, acc, sg0, sg1, ss0, ss1):
  del ss0, ss1
  pltpu.async_copy(h_hbm.at[sv.at[0]], ga, sg0)

  @pl.loop(0, nch // 2 - 1)
  def _(i):
    j = 2 * i
    pltpu.async_copy(h_hbm.at[sv.at[j + 1]], gb, sg1)
    pltpu.make_async_copy(h_hbm.at[sv.at[j]], ga, sg0).wait()
    pltpu.sync_copy(ga, acc.at[dv.at[j]], add=True)
    pltpu.async_copy(h_hbm.at[sv.at[j + 2]], ga, sg0)
    pltpu.make_async_copy(h_hbm.at[sv.at[j + 1]], gb, sg1).wait()
    pltpu.sync_copy(gb, acc.at[dv.at[j + 1]], add=True)

  pltpu.async_copy(h_hbm.at[sv.at[nch - 1]], gb, sg1)
  pltpu.make_async_copy(h_hbm.at[sv.at[nch - 2]], ga, sg0).wait()
  pltpu.sync_copy(ga, acc.at[dv.at[nch - 2]], add=True)
  pltpu.make_async_copy(h_hbm.at[sv.at[nch - 1]], gb, sg1).wait()
  pltpu.sync_copy(gb, acc.at[dv.at[nch - 1]], add=True)


def _zero_acc(gbuf, acc, s):
  zv = jnp.zeros((16,), jnp.float32)

  @pl.loop(0, 16)
  def _(r):
    for j in range(8):
      gbuf[r, pl.ds(j * 16, 16)] = zv

  @pl.loop(0, RPT // 16)
  def _(t):
    pltpu.sync_copy(gbuf.at[pl.ds(0, 16)], acc.at[pl.ds(s * RPT + t * 16, 16)])


@functools.partial(
    pl.kernel,
    out_type=jax.ShapeDtypeStruct((2, NPAD, 128), jnp.float32),
    mesh=_MESH,
    scratch_types=[
        pltpu.VMEM((GN, CW), jnp.int32),
        pltpu.VMEM((GN, CW), jnp.int32),
        pltpu.VMEM((CW, 128), jnp.float32),
        pltpu.VMEM((CW, 128), jnp.float32),
        pltpu.VMEM_SHARED((NPAD, 128), jnp.float32),
        pltpu.SemaphoreType.DMA,
        pltpu.SemaphoreType.DMA,
        pltpu.SemaphoreType.DMA,
        pltpu.SemaphoreType.DMA,
    ],
    name="agg1",
)
def _agg1(h_hbm, src_hbm, dst_hbm, out_hbm, src_v, dst_v, ga, gb, acc,
          sg0, sg1, ss0, ss1):
  c = lax.axis_index("c")
  s = lax.axis_index("s")
  _zero_acc(ga, acc, s)
  plsc.subcore_barrier()
  for g in range(NCH // GN):
    pltpu.sync_copy(src_hbm.at[c, s, g], src_v)
    pltpu.sync_copy(dst_hbm.at[s, g], dst_v)
    _edge_pipe(GN, h_hbm, src_v, dst_v, ga, gb, acc, sg0, sg1, ss0, ss1)
  plsc.subcore_barrier()
  pltpu.sync_copy(acc.at[pl.ds(s * RPT, RPT)],
                  out_hbm.at[c, pl.ds(s * RPT, RPT)])


@functools.partial(
    pl.kernel,
    out_type=jax.ShapeDtypeStruct((2, NPAD, 128), jnp.float32),
    mesh=_MESH,
    scratch_types=[
        pltpu.VMEM((NCH2, CW), jnp.int32),
        pltpu.VMEM((NCH2, CW), jnp.int32),
        pltpu.VMEM((CW, 128), jnp.float32),
        pltpu.VMEM((CW, 128), jnp.float32),
        pltpu.VMEM_SHARED((NPAD, 128), jnp.float32),
        pltpu.SemaphoreType.DMA,
        pltpu.SemaphoreType.DMA,
        pltpu.SemaphoreType.DMA,
        pltpu.SemaphoreType.DMA,
    ],
    name="agg2",
)
def _agg2(h_hbm, src_hbm, dst_hbm, out_hbm, src_v, dst_v, ga, gb, acc,
          sg0, sg1, ss0, ss1):
  c = lax.axis_index("c")
  s = lax.axis_index("s")
  w = c * 16 + s
  _zero_acc(ga, acc, s)
  pltpu.sync_copy(src_hbm.at[w], src_v)
  pltpu.sync_copy(dst_hbm.at[w], dst_v)
  plsc.subcore_barrier()
  _edge_pipe(NCH2, h_hbm, src_v, dst_v, ga, gb, acc, sg0, sg1, ss0, ss1)
  plsc.subcore_barrier()
  pltpu.sync_copy(acc.at[pl.ds(s * RPT, RPT)],
                  out_hbm.at[c, pl.ds(s * RPT, RPT)])


@functools.partial(
    pl.kernel,
    out_type=jax.ShapeDtypeStruct((2, NPAD, 128), jnp.float32),
    mesh=_MESH,
    scratch_types=[
        pltpu.VMEM((DCH, DCW), jnp.int32),
        pltpu.VMEM((DCW, 128), jnp.float32),
        pltpu.VMEM_SHARED((NPAD, 128), jnp.float32),
    ],
    name="deg",
)
def _deg(dst_hbm, out_hbm, dst_v, obuf, acc):
  c = lax.axis_index("c")
  s = lax.axis_index("s")
  w = c * 16 + s
  _zero_acc(obuf, acc, s)
  ones = jnp.ones((16,), jnp.float32)

  @pl.loop(0, DCW)
  def _(r):
    for j in range(8):
      obuf[r, pl.ds(j * 16, 16)] = ones

  pltpu.sync_copy(dst_hbm.at[w], dst_v)
  plsc.subcore_barrier()

  @pl.loop(0, DCH)
  def _(j):
    pltpu.sync_copy(obuf, acc.at[dst_v.at[j]], add=True)

  plsc.subcore_barrier()
  pltpu.sync_copy(acc.at[pl.ds(s * RPT, RPT)],
                  out_hbm.at[c, pl.ds(s * RPT, RPT)])


def _dinv_body(degp_ref, out_ref):
  total = degp_ref[0] + degp_ref[1]
  out_ref[...] = lax.rsqrt(total[:, 0:1])


def _dinv(degp):
  return pl.pallas_call(
      _dinv_body,
      out_shape=jax.ShapeDtypeStruct((NPAD, 1), jnp.float32),
  )(degp)


def _mm1_body(x_ref, w_ref, dinv_ref, out_ref):
  h = jnp.dot(x_ref[...], w_ref[...], preferred_element_type=jnp.float32)
  out_ref[0] = h * dinv_ref[...]


def _mm1(x_pooled, W1, dinv_p):
  blk = 1000
  return pl.pallas_call(
      _mm1_body,
      grid=(2, NP // blk),
      in_specs=[
          pl.BlockSpec((blk, D), lambda c, r: (r, 0)),
          pl.BlockSpec((D, 128), lambda c, r: (0, c)),
          pl.BlockSpec((blk, 1), lambda c, r: (r, 0)),
      ],
      out_specs=pl.BlockSpec((1, blk, 128), lambda c, r: (c, r, 0)),
      out_shape=jax.ShapeDtypeStruct((2, NP, 128), jnp.float32),
  )(x_pooled, W1, dinv_p)


def _tcb_body(a0_ref, a1_ref, dinv_ref, b1_ref, g1_ref, bt_ref, wf_ref,
              h2_ref):
  agg = jnp.concatenate([a0_ref[0], a1_ref[0]], axis=-1)
  dinv = dinv_ref[...]
  z = agg * dinv + b1_ref[...]
  a = jnp.maximum(z * (g1_ref[...] * BN_K) + bt_ref[...], 0.0)
  h2_ref[...] = jnp.dot(a, wf_ref[...],
                        preferred_element_type=jnp.float32) * dinv


def _tcb(agg1, dinv, b1, gamma1, beta1, Wf):
  blk = 1000
  return pl.pallas_call(
      _tcb_body,
      grid=(N // blk,),
      in_specs=[
          pl.BlockSpec((1, blk, 128), lambda r: (0, r, 0)),
          pl.BlockSpec((1, blk, 128), lambda r: (1, r, 0)),
          pl.BlockSpec((blk, 1), lambda r: (r, 0)),
          pl.BlockSpec((1, DH), lambda r: (0, 0)),
          pl.BlockSpec((1, DH), lambda r: (0, 0)),
          pl.BlockSpec((1, DH), lambda r: (0, 0)),
          pl.BlockSpec((DH, DO), lambda r: (0, 0)),
      ],
      out_specs=pl.BlockSpec((blk, DO), lambda r: (r, 0)),
      out_shape=jax.ShapeDtypeStruct((N, DO), jnp.float32),
  )(agg1, agg1, dinv, b1, gamma1, beta1, Wf)


def _tcc_body(p0_ref, p1_ref, dinv_ref, bf_ref, out_ref):
  agg = p0_ref[0] + p1_ref[0]
  out_ref[...] = agg * dinv_ref[...] + bf_ref[...]


def _tcc(agg2, dinv, bf):
  blk = 1000
  return pl.pallas_call(
      _tcc_body,
      grid=(N // blk,),
      in_specs=[
          pl.BlockSpec((1, blk, 128), lambda r: (0, r, 0)),
          pl.BlockSpec((1, blk, 128), lambda r: (1, r, 0)),
          pl.BlockSpec((blk, 1), lambda r: (r, 0)),
          pl.BlockSpec((1, DO), lambda r: (0, 0)),
      ],
      out_specs=pl.BlockSpec((blk, DO), lambda r: (r, 0)),
      out_shape=jax.ShapeDtypeStruct((N, DO), jnp.float32),
  )(agg2, agg2, dinv, bf)


def kernel(x_pooled, edge_index_latent, batch_latent, perm, edge_index_full,
           batch_full, num_nodes_before_pool, W1, b1, gamma1, beta1, Wf, bf):
  src = edge_index_full[0].astype(jnp.int32)
  dst = edge_index_full[1].astype(jnp.int32)
  loops = jnp.arange(N, dtype=jnp.int32)
  src_l = jnp.concatenate([src, loops])
  dst_l = jnp.concatenate([dst, loops])
  src_p1 = jnp.concatenate(
      [src_l, jnp.full((EP - EL,), NP, jnp.int32)]).reshape(
          16, NCH // GN, GN, CW)
  src2 = jnp.stack([src_p1, src_p1 + N])
  src_c2 = jnp.concatenate(
      [src_l, jnp.full((EP - EL,), N, jnp.int32)]).reshape(32, NCH2, CW)
  dst_p = jnp.concatenate(
      [dst_l, jnp.full((EP - EL,), N, jnp.int32)])
  dst16 = dst_p.reshape(16, NCH // GN, GN, CW)
  dst32 = dst_p.reshape(32, NCH2, CW)
  dst_deg = jnp.concatenate(
      [dst_l, jnp.full((EPD - EL,), N, jnp.int32)]).reshape(32, DCH, DCW)

  degp = _deg(dst_deg)
  dinv = _dinv(degp)

  hs = _mm1(x_pooled, W1, dinv[:NP])
  ht1 = jnp.concatenate(
      [hs, jnp.zeros((2, NP, 128), jnp.float32)], axis=1).reshape(2 * N, 128)

  agg1 = _agg1(ht1, src2, dst16)

  h2 = _tcb(agg1, dinv[:N], b1.reshape(1, DH), gamma1.reshape(1, DH),
            beta1.reshape(1, DH), Wf)
  ht2 = jnp.concatenate([h2, jnp.zeros((8, DO), jnp.float32)], axis=0)

  agg2 = _agg2(ht2, src_c2, dst32)

  out = _tcc(agg2, dinv[:N], bf.reshape(1, DO))
  return out, batch_full


# speedup vs baseline: 2.8385x; 2.7479x over previous
import functools
import math

import jax
import jax.numpy as jnp
from jax import lax
from jax.experimental import pallas as pl
from jax.experimental.pallas import tpu as pltpu
from jax.experimental.pallas import tpu_sc as plsc

N = 10000
NP = 5000
D = 256
DH = 256
DO = 128
E = 160000
EL = E + N
NPAD = 10240
RPT = NPAD // 16
CW = 125
GN = 44
NCH = 88
EP = 16 * NCH * CW
NCH2 = 44
DCH = 42
DCW = 128
EPD = 32 * DCH * DCW
BN_K = 1.0 / math.sqrt(1.0 + 1e-5)

_MESH = plsc.VectorSubcoreMesh(core_axis_name="c", subcore_axis_name="s")


def _edge_pipe(nch, h_hbm, sv, dv, ga, gb, acc, sg0, sg1, ss0, ss1):
  del ss0, ss1
  pltpu.async_copy(h_hbm.at[sv.at[0]], ga, sg0)

  @pl.loop(0, nch // 2 - 1)
  def _(i):
    j = 2 * i
    pltpu.async_copy(h_hbm.at[sv.at[j + 1]], gb, sg1)
    pltpu.make_async_copy(h_hbm.at[sv.at[j]], ga, sg0).wait()
    pltpu.sync_copy(ga, acc.at[dv.at[j]], add=True)
    pltpu.async_copy(h_hbm.at[sv.at[j + 2]], ga, sg0)
    pltpu.make_async_copy(h_hbm.at[sv.at[j + 1]], gb, sg1).wait()
    pltpu.sync_copy(gb, acc.at[dv.at[j + 1]], add=True)

  pltpu.async_copy(h_hbm.at[sv.at[nch - 1]], gb, sg1)
  pltpu.make_async_copy(h_hbm.at[sv.at[nch - 2]], ga, sg0).wait()
  pltpu.sync_copy(ga, acc.at[dv.at[nch - 2]], add=True)
  pltpu.make_async_copy(h_hbm.at[sv.at[nch - 1]], gb, sg1).wait()
  pltpu.sync_copy(gb, acc.at[dv.at[nch - 1]], add=True)


def _zero_acc(gbuf, acc, s):
  zv = jnp.zeros((16,), jnp.float32)

  @pl.loop(0, 16)
  def _(r):
    for j in range(8):
      gbuf[r, pl.ds(j * 16, 16)] = zv

  @pl.loop(0, RPT // 16)
  def _(t):
    pltpu.sync_copy(gbuf.at[pl.ds(0, 16)], acc.at[pl.ds(s * RPT + t * 16, 16)])


@functools.partial(
    pl.kernel,
    out_type=jax.ShapeDtypeStruct((2, NPAD, 128), jnp.float32),
    mesh=_MESH,
    scratch_types=[
        pltpu.VMEM((GN, CW), jnp.int32),
        pltpu.VMEM((GN, CW), jnp.int32),
        pltpu.VMEM((CW, 128), jnp.float32),
        pltpu.VMEM((CW, 128), jnp.float32),
        pltpu.VMEM_SHARED((NPAD, 128), jnp.float32),
        pltpu.SemaphoreType.DMA,
        pltpu.SemaphoreType.DMA,
        pltpu.SemaphoreType.DMA,
        pltpu.SemaphoreType.DMA,
    ],
    name="agg1",
)
def _agg1(h_hbm, src_hbm, dst_hbm, out_hbm, src_v, dst_v, ga, gb, acc,
          sg0, sg1, ss0, ss1):
  c = lax.axis_index("c")
  s = lax.axis_index("s")
  _zero_acc(ga, acc, s)
  plsc.subcore_barrier()
  for g in range(NCH // GN):
    pltpu.sync_copy(src_hbm.at[c, s, g], src_v)
    pltpu.sync_copy(dst_hbm.at[s, g], dst_v)
    _edge_pipe(GN, h_hbm, src_v, dst_v, ga, gb, acc, sg0, sg1, ss0, ss1)
  plsc.subcore_barrier()
  pltpu.sync_copy(acc.at[pl.ds(s * RPT, RPT)],
                  out_hbm.at[c, pl.ds(s * RPT, RPT)])


@functools.partial(
    pl.kernel,
    out_type=jax.ShapeDtypeStruct((2, NPAD, 128), jnp.float32),
    mesh=_MESH,
    scratch_types=[
        pltpu.VMEM((NCH2, CW), jnp.int32),
        pltpu.VMEM((NCH2, CW), jnp.int32),
        pltpu.VMEM((CW, 128), jnp.float32),
        pltpu.VMEM((CW, 128), jnp.float32),
        pltpu.VMEM_SHARED((NPAD, 128), jnp.float32),
        pltpu.SemaphoreType.DMA,
        pltpu.SemaphoreType.DMA,
        pltpu.SemaphoreType.DMA,
        pltpu.SemaphoreType.DMA,
    ],
    name="agg2",
)
def _agg2(h_hbm, src_hbm, dst_hbm, out_hbm, src_v, dst_v, ga, gb, acc,
          sg0, sg1, ss0, ss1):
  c = lax.axis_index("c")
  s = lax.axis_index("s")
  w = c * 16 + s
  _zero_acc(ga, acc, s)
  pltpu.sync_copy(src_hbm.at[w], src_v)
  pltpu.sync_copy(dst_hbm.at[w], dst_v)
  plsc.subcore_barrier()
  _edge_pipe(NCH2, h_hbm, src_v, dst_v, ga, gb, acc, sg0, sg1, ss0, ss1)
  plsc.subcore_barrier()
  pltpu.sync_copy(acc.at[pl.ds(s * RPT, RPT)],
                  out_hbm.at[c, pl.ds(s * RPT, RPT)])


@functools.partial(
    pl.kernel,
    out_type=jax.ShapeDtypeStruct((2, NPAD, 128), jnp.float32),
    mesh=_MESH,
    scratch_types=[
        pltpu.VMEM((DCH, DCW), jnp.int32),
        pltpu.VMEM((DCW, 128), jnp.float32),
        pltpu.VMEM_SHARED((NPAD, 128), jnp.float32),
    ],
    name="deg",
)
def _deg(dst_hbm, out_hbm, dst_v, obuf, acc):
  c = lax.axis_index("c")
  s = lax.axis_index("s")
  w = c * 16 + s
  _zero_acc(obuf, acc, s)
  ones = jnp.ones((16,), jnp.float32)

  @pl.loop(0, DCW)
  def _(r):
    for j in range(8):
      obuf[r, pl.ds(j * 16, 16)] = ones

  pltpu.sync_copy(dst_hbm.at[w], dst_v)
  plsc.subcore_barrier()

  @pl.loop(0, DCH)
  def _(j):
    pltpu.sync_copy(obuf, acc.at[dst_v.at[j]], add=True)

  plsc.subcore_barrier()
  pltpu.sync_copy(acc.at[pl.ds(s * RPT, RPT)],
                  out_hbm.at[c, pl.ds(s * RPT, RPT)])


def _dinv_body(degp_ref, out_ref):
  total = degp_ref[0] + degp_ref[1]
  out_ref[...] = lax.rsqrt(total[:, 0:1])


def _dinv(degp):
  return pl.pallas_call(
      _dinv_body,
      out_shape=jax.ShapeDtypeStruct((NPAD, 1), jnp.float32),
  )(degp)


def _mm1_body(x_ref, w_ref, dinv_ref, out_ref):
  h = jnp.dot(x_ref[...], w_ref[...], preferred_element_type=jnp.float32)
  out_ref[0] = h * dinv_ref[...]


def _mm1(x_pooled, W1, dinv_p):
  blk = 1000
  return pl.pallas_call(
      _mm1_body,
      grid=(2, NP // blk),
      in_specs=[
          pl.BlockSpec((blk, D), lambda c, r: (r, 0)),
          pl.BlockSpec((D, 128), lambda c, r: (0, c)),
          pl.BlockSpec((blk, 1), lambda c, r: (r, 0)),
      ],
      out_specs=pl.BlockSpec((1, blk, 128), lambda c, r: (c, r, 0)),
      out_shape=jax.ShapeDtypeStruct((2, NP, 128), jnp.float32),
  )(x_pooled, W1, dinv_p)


def _tcb_body(a0_ref, a1_ref, dinv_ref, b1_ref, g1_ref, bt_ref, wf_ref,
              h2_ref):
  agg = jnp.concatenate([a0_ref[0], a1_ref[0]], axis=-1)
  dinv = dinv_ref[...]
  z = agg * dinv + b1_ref[...]
  a = jnp.maximum(z * (g1_ref[...] * BN_K) + bt_ref[...], 0.0)
  h2_ref[...] = jnp.dot(a, wf_ref[...],
                        preferred_element_type=jnp.float32) * dinv


def _tcb(agg1, dinv, b1, gamma1, beta1, Wf):
  blk = 1000
  return pl.pallas_call(
      _tcb_body,
      grid=(N // blk,),
      in_specs=[
          pl.BlockSpec((1, blk, 128), lambda r: (0, r, 0)),
          pl.BlockSpec((1, blk, 128), lambda r: (1, r, 0)),
          pl.BlockSpec((blk, 1), lambda r: (r, 0)),
          pl.BlockSpec((1, DH), lambda r: (0, 0)),
          pl.BlockSpec((1, DH), lambda r: (0, 0)),
          pl.BlockSpec((1, DH), lambda r: (0, 0)),
          pl.BlockSpec((DH, DO), lambda r: (0, 0)),
      ],
      out_specs=pl.BlockSpec((blk, DO), lambda r: (r, 0)),
      out_shape=jax.ShapeDtypeStruct((N, DO), jnp.float32),
  )(agg1, agg1, dinv, b1, gamma1, beta1, Wf)


def _tcc_body(p0_ref, p1_ref, dinv_ref, bf_ref, out_ref):
  agg = p0_ref[0] + p1_ref[0]
  out_ref[...] = agg * dinv_ref[...] + bf_ref[...]


def _tcc(agg2, dinv, bf):
  blk = 1000
  return pl.pallas_call(
      _tcc_body,
      grid=(N // blk,),
      in_specs=[
          pl.BlockSpec((1, blk, 128), lambda r: (0, r, 0)),
          pl.BlockSpec((1, blk, 128), lambda r: (1, r, 0)),
          pl.BlockSpec((blk, 1), lambda r: (r, 0)),
          pl.BlockSpec((1, DO), lambda r: (0, 0)),
      ],
      out_specs=pl.BlockSpec((blk, DO), lambda r: (r, 0)),
      out_shape=jax.ShapeDtypeStruct((N, DO), jnp.float32),
  )(agg2, agg2, dinv, bf)


def kernel(x_pooled, edge_index_latent, batch_latent, perm, edge_index_full,
           batch_full, num_nodes_before_pool, W1, b1, gamma1, beta1, Wf, bf):
  src = edge_index_full[0].astype(jnp.int32)
  dst = edge_index_full[1].astype(jnp.int32)
  loops = jnp.arange(N, dtype=jnp.int32)
  src_l = jnp.concatenate([src, loops])
  dst_l = jnp.concatenate([dst, loops])
  pad_ar = jnp.arange(EP - EL, dtype=jnp.int32)
  src_p1 = jnp.concatenate(
      [src_l, NP + pad_ar % 1000]).reshape(16, NCH // GN, GN, CW)
  src2 = jnp.stack([src_p1, src_p1 + N])
  src_c2 = jnp.concatenate(
      [src_l, N + pad_ar % 8]).reshape(32, NCH2, CW)
  dst_p = jnp.concatenate([dst_l, N + pad_ar % (NPAD - N)])
  dst16 = dst_p.reshape(16, NCH // GN, GN, CW)
  dst32 = dst_p.reshape(32, NCH2, CW)
  pad_dr = jnp.arange(EPD - EL, dtype=jnp.int32)
  dst_deg = jnp.concatenate(
      [dst_l, N + pad_dr % (NPAD - N)]).reshape(32, DCH, DCW)

  degp = _deg(dst_deg)
  dinv = _dinv(degp)

  hs = _mm1(x_pooled, W1, dinv[:NP])
  ht1 = jnp.concatenate(
      [hs, jnp.zeros((2, NP, 128), jnp.float32)], axis=1).reshape(2 * N, 128)

  agg1 = _agg1(ht1, src2, dst16)

  h2 = _tcb(agg1, dinv[:N], b1.reshape(1, DH), gamma1.reshape(1, DH),
            beta1.reshape(1, DH), Wf)
  ht2 = jnp.concatenate([h2, jnp.zeros((8, DO), jnp.float32)], axis=0)

  agg2 = _agg2(ht2, src_c2, dst32)

  out = _tcc(agg2, dinv[:N], bf.reshape(1, DO))
  return out, batch_full
